# baseline (device time: 34003 ns/iter reference)
import jax
import jax.numpy as jnp
from jax import lax
from jax.experimental import pallas as pl
from jax.experimental.pallas import tpu as pltpu

N_DEV = 4
B = 2
SQ = 128
SKV = 128
D = 512
HQ_LOCAL = 8
HKV_LOCAL = 2
GROUP = 4
DH = 64
SCALE = 0.125


def kernel(x, Wq, Wo, K_ext, V_ext):
    my_pos = lax.axis_index("i")
    k_loc = lax.dynamic_slice_in_dim(K_ext, my_pos * HKV_LOCAL, HKV_LOCAL, axis=2)
    v_loc = lax.dynamic_slice_in_dim(V_ext, my_pos * HKV_LOCAL, HKV_LOCAL, axis=2)
    k_loc = jnp.transpose(k_loc, (0, 2, 1, 3))
    v_loc = jnp.transpose(v_loc, (0, 2, 1, 3))

    def body(x_ref, wq_ref, wo_ref, k_ref, v_ref, out_ref,
             comm_ref, send_sems, recv_sems):
        me = lax.axis_index("i")
        left = lax.rem(me + N_DEV - 1, N_DEV)
        right = lax.rem(me + 1, N_DEV)

        barrier_sem = pltpu.get_barrier_semaphore()
        for nbr in (left, right):
            pl.semaphore_signal(
                barrier_sem, inc=1,
                device_id=(nbr,), device_id_type=pl.DeviceIdType.MESH,
            )
        pl.semaphore_wait(barrier_sem, 2)

        x2 = x_ref[...].reshape(B * SQ, D)
        q2 = jnp.dot(x2, wq_ref[...], preferred_element_type=jnp.float32)

        head_outs = []
        for b in range(B):
            rows = []
            for h in range(HQ_LOCAL):
                q_h = q2[b * SQ:(b + 1) * SQ, h * DH:(h + 1) * DH]
                k_h = k_ref[b, h // GROUP, :, :]
                v_h = v_ref[b, h // GROUP, :, :]
                s = lax.dot_general(
                    q_h, k_h, (((1,), (1,)), ((), ())),
                    preferred_element_type=jnp.float32,
                ) * SCALE
                m = jnp.max(s, axis=1, keepdims=True)
                p = jnp.exp(s - m)
                l = jnp.sum(p, axis=1, keepdims=True)
                o = jnp.dot(p, v_h, preferred_element_type=jnp.float32) / l
                rows.append(o)
            head_outs.append(jnp.concatenate(rows, axis=1))
        attn2 = jnp.concatenate(head_outs, axis=0)

        partial = jnp.dot(attn2, wo_ref[...], preferred_element_type=jnp.float32)

        acc = partial
        comm_ref[0, :, :] = partial
        for hop in range(N_DEV - 1):
            send_slot = hop % 2
            recv_slot = (hop + 1) % 2
            rdma = pltpu.make_async_remote_copy(
                src_ref=comm_ref.at[send_slot],
                dst_ref=comm_ref.at[recv_slot],
                send_sem=send_sems.at[send_slot],
                recv_sem=recv_sems.at[recv_slot],
                device_id=(right,),
                device_id_type=pl.DeviceIdType.MESH,
            )
            rdma.start()
            rdma.wait()
            acc = acc + comm_ref[recv_slot, :, :]

        out_ref[...] = acc.reshape(B, SQ, D)

    return pl.pallas_call(
        body,
        out_shape=jax.ShapeDtypeStruct((B, SQ, D), jnp.float32),
        in_specs=[pl.BlockSpec(memory_space=pltpu.VMEM)] * 5,
        out_specs=pl.BlockSpec(memory_space=pltpu.VMEM),
        scratch_shapes=[
            pltpu.VMEM((2, B * SQ, D), jnp.float32),
            pltpu.SemaphoreType.DMA((2,)),
            pltpu.SemaphoreType.DMA((2,)),
        ],
        compiler_params=pltpu.CompilerParams(collective_id=0),
    )(x, Wq, Wo, k_loc, v_loc)


# device time: 24772 ns/iter; 1.3726x vs baseline; 1.3726x over previous
import jax
import jax.numpy as jnp
from jax import lax
from jax.experimental import pallas as pl
from jax.experimental.pallas import tpu as pltpu

N_DEV = 4
B = 2
SQ = 128
SKV = 128
D = 512
HQ_LOCAL = 8
HKV_LOCAL = 2
GROUP = 4
DH = 64
SCALE = 0.125

HALF = B * SQ // 2
QTR = B * SQ // 4


def kernel(x, Wq, Wo, K_ext, V_ext):
    my_pos = lax.axis_index("i")
    k_loc = lax.dynamic_slice_in_dim(K_ext, my_pos * HKV_LOCAL, HKV_LOCAL, axis=2)
    v_loc = lax.dynamic_slice_in_dim(V_ext, my_pos * HKV_LOCAL, HKV_LOCAL, axis=2)
    k_loc = jnp.transpose(k_loc, (0, 2, 1, 3))
    v_loc = jnp.transpose(v_loc, (0, 2, 1, 3))

    def body(x_ref, wq_ref, wo_ref, k_ref, v_ref, out_ref,
             work_ref, half_recv, q_recv, send_sems, recv_sems):
        me = lax.axis_index("i")
        xc = me // 2
        yc = (me ^ (me >> 1)) & 1
        p1 = 3 - me
        p2 = me ^ 1

        barrier_sem = pltpu.get_barrier_semaphore()
        for nbr in (p1, p2):
            pl.semaphore_signal(
                barrier_sem, inc=1,
                device_id=(nbr,), device_id_type=pl.DeviceIdType.MESH,
            )
        pl.semaphore_wait(barrier_sem, 2)

        keep = xc * HALF
        give = (1 - xc) * HALF
        qkeep = keep + yc * QTR
        qgive = keep + (1 - yc) * QTR
        b_give = 1 - xc
        b_keep = xc

        def partial_for_batch(b):
            xb = x_ref[b]
            qb = jnp.dot(xb, wq_ref[...],
                         preferred_element_type=jnp.float32)
            outs = []
            for h in range(HQ_LOCAL):
                q_h = qb[:, h * DH:(h + 1) * DH]
                k_h = k_ref[b, h // GROUP]
                v_h = v_ref[b, h // GROUP]
                s = lax.dot_general(
                    q_h, k_h, (((1,), (1,)), ((), ())),
                    preferred_element_type=jnp.float32,
                ) * SCALE
                m = jnp.max(s, axis=1, keepdims=True)
                p = jnp.exp(s - m)
                l = jnp.sum(p, axis=1, keepdims=True)
                outs.append(jnp.dot(p, v_h, preferred_element_type=jnp.float32) / l)
            attn = jnp.concatenate(outs, axis=1)
            return jnp.dot(attn, wo_ref[...],
                           preferred_element_type=jnp.float32)

        work_ref[pl.ds(give, HALF), :] = partial_for_batch(b_give)
        rs1 = pltpu.make_async_remote_copy(
            src_ref=work_ref.at[pl.ds(give, HALF)],
            dst_ref=half_recv,
            send_sem=send_sems.at[0],
            recv_sem=recv_sems.at[0],
            device_id=(p1,),
            device_id_type=pl.DeviceIdType.MESH,
        )
        rs1.start()
        work_ref[pl.ds(keep, HALF), :] = partial_for_batch(b_keep)
        rs1.wait()
        work_ref[pl.ds(keep, HALF), :] = (
            work_ref[pl.ds(keep, HALF), :] + half_recv[...]
        )

        rs2 = pltpu.make_async_remote_copy(
            src_ref=work_ref.at[pl.ds(qgive, QTR)],
            dst_ref=q_recv,
            send_sem=send_sems.at[1],
            recv_sem=recv_sems.at[1],
            device_id=(p2,),
            device_id_type=pl.DeviceIdType.MESH,
        )
        rs2.start()
        rs2.wait()
        work_ref[pl.ds(qkeep, QTR), :] = (
            work_ref[pl.ds(qkeep, QTR), :] + q_recv[...]
        )

        ag2 = pltpu.make_async_remote_copy(
            src_ref=work_ref.at[pl.ds(qkeep, QTR)],
            dst_ref=work_ref.at[pl.ds(qkeep, QTR)],
            send_sem=send_sems.at[2],
            recv_sem=recv_sems.at[2],
            device_id=(p2,),
            device_id_type=pl.DeviceIdType.MESH,
        )
        ag2.start()
        ag2.wait()

        ag1 = pltpu.make_async_remote_copy(
            src_ref=work_ref.at[pl.ds(keep, HALF)],
            dst_ref=work_ref.at[pl.ds(keep, HALF)],
            send_sem=send_sems.at[3],
            recv_sem=recv_sems.at[3],
            device_id=(p1,),
            device_id_type=pl.DeviceIdType.MESH,
        )
        ag1.start()
        ag1.wait()

        out_ref[...] = work_ref[...].reshape(B, SQ, D)

    return pl.pallas_call(
        body,
        out_shape=jax.ShapeDtypeStruct((B, SQ, D), jnp.float32),
        in_specs=[pl.BlockSpec(memory_space=pltpu.VMEM)] * 5,
        out_specs=pl.BlockSpec(memory_space=pltpu.VMEM),
        scratch_shapes=[
            pltpu.VMEM((B * SQ, D), jnp.float32),
            pltpu.VMEM((HALF, D), jnp.float32),
            pltpu.VMEM((QTR, D), jnp.float32),
            pltpu.SemaphoreType.DMA((4,)),
            pltpu.SemaphoreType.DMA((4,)),
        ],
        compiler_params=pltpu.CompilerParams(collective_id=0),
    )(x, Wq, Wo, k_loc, v_loc)
